# baseline (speedup 1.0000x reference)
import functools

import jax
import jax.numpy as jnp
from jax import lax
from jax.experimental import pallas as pl
from jax.experimental.pallas import tpu as pltpu
from jax.experimental.pallas import tpu_sc as plsc

NCLS = 19
NB = 4
H = W = 512
NC, NS, L = 2, 16, 16
NW = NC * NS
ROWS = 4

TC_H = 192
SC_H = H - TC_H

BANDS = 16
WCOL = W // 2
BAND_H = SC_H // BANDS
CHUNKS_PER_B = BAND_H // ROWS
TOTAL_CHUNKS = NB * CHUNKS_PER_B
CHUNK = ROWS * WCOL
STEPS = CHUNK // L

TC_RB = 16


def _sc_body(preds_hbm, target_hbm, out_hbm, pbuf, tbuf, hp, ht, hi, part,
             sp0, sp1, st0, st1):
    wid = lax.axis_index("s") * NC + lax.axis_index("c")
    band = lax.shift_right_logical(wid, 1)
    col0 = (wid & 1) * WCOL
    psems = (sp0, sp1)
    tsems = (st0, st1)

    def _descr(k, j):
        if isinstance(k, int):
            b = k // CHUNKS_PER_B
            rem = k % CHUNKS_PER_B
        else:
            b = lax.shift_right_logical(k * 13, 6)
            rem = k - b * CHUNKS_PER_B
        row0 = TC_H + band * BAND_H + rem * ROWS
        dp = pltpu.make_async_copy(
            preds_hbm.at[b, :, pl.ds(row0, ROWS), pl.ds(col0, WCOL)],
            pbuf.at[j], psems[j])
        dt = pltpu.make_async_copy(
            target_hbm.at[b, pl.ds(row0, ROWS), pl.ds(col0, WCOL)],
            tbuf.at[j], tsems[j])
        return dp, dt

    z = jnp.zeros((L,), jnp.float32)
    for r in range(L):
        for h in (hp, ht, hi):
            h[r, pl.ds(0, L)] = z
            h[r, pl.ds(L, L)] = z

    lane = lax.iota(jnp.int32, L)
    ones = jnp.ones((L,), jnp.float32)

    def _argmax_tree(vals):
        idxs = [jnp.full((L,), c, jnp.int32) for c in range(NCLS)]
        while len(vals) > 1:
            nv, ni = [], []
            for a in range(0, len(vals) - 1, 2):
                g = vals[a + 1] > vals[a]
                nv.append(jnp.where(g, vals[a + 1], vals[a]))
                ni.append(jnp.where(g, idxs[a + 1], idxs[a]))
            if len(vals) % 2:
                nv.append(vals[-1])
                ni.append(idxs[-1])
            vals, idxs = nv, ni
        return idxs[0]

    BLOCK = STEPS // ROWS

    def _compute(j):
        def blk(r, carry):
            vals = [pbuf[j, c, r, pl.ds(0, L)] for c in range(NCLS)]
            tcur = tbuf[j, r, pl.ds(0, L)]
            vals1 = [pbuf[j, c, r, pl.ds(L, L)] for c in range(NCLS)]
            tnxt = tbuf[j, r, pl.ds(L, L)]
            for s in range(BLOCK):
                idx = _argmax_tree(vals)
                eq = idx == tcur
                tprev = tcur
                vals, tcur = vals1, tnxt
                if s + 2 < BLOCK:
                    nb = (s + 2) * L
                    vals1 = [pbuf[j, c, r, pl.ds(nb, L)] for c in range(NCLS)]
                    tnxt = tbuf[j, r, pl.ds(nb, L)]
                plsc.addupdate_scatter(hp, [lane, idx], ones)
                plsc.addupdate_scatter(ht, [lane, tprev], ones)
                plsc.addupdate_scatter(hi, [lane, tprev], ones, mask=eq)
            return carry

        lax.fori_loop(0, ROWS, blk, 0)

    d0p, d0t = _descr(0, 0)
    d0p.start()
    d0t.start()
    d1p, d1t = _descr(1, 1)
    d1p.start()
    d1t.start()

    def pair(p, carry):
        k0 = p * 2
        for j in (0, 1):
            k = k0 + j
            dp, dt = _descr(k, j)
            dp.wait()
            dt.wait()
            _compute(j)

            @pl.when(k + 2 < TOTAL_CHUNKS)
            def _():
                sp, st = _descr(k + 2, j)
                sp.start()
                st.start()
        return carry

    lax.fori_loop(0, TOTAL_CHUNKS // 2, pair, 0)

    for row, h in enumerate((hp, ht, hi)):
        a0 = h[0, pl.ds(0, L)]
        a1 = h[0, pl.ds(L, L)]
        for r in range(1, L):
            a0 = a0 + h[r, pl.ds(0, L)]
            a1 = a1 + h[r, pl.ds(L, L)]
        part[row, pl.ds(0, L)] = a0
        part[row, pl.ds(L, L)] = a1
    pltpu.sync_copy(part, out_hbm.at[wid])


_sc_hist = functools.partial(
    pl.kernel,
    out_type=jax.ShapeDtypeStruct((NW, 3, 32), jnp.float32),
    mesh=plsc.VectorSubcoreMesh(core_axis_name="c", subcore_axis_name="s"),
    compiler_params=pltpu.CompilerParams(
        needs_layout_passes=False, skip_device_barrier=True),
    scratch_types=[
        pltpu.VMEM((2, NCLS, ROWS, WCOL), jnp.float32),
        pltpu.VMEM((2, ROWS, WCOL), jnp.int32),
        pltpu.VMEM((L, 32), jnp.float32),
        pltpu.VMEM((L, 32), jnp.float32),
        pltpu.VMEM((L, 32), jnp.float32),
        pltpu.VMEM((3, 32), jnp.float32),
        pltpu.SemaphoreType.DMA,
        pltpu.SemaphoreType.DMA,
        pltpu.SemaphoreType.DMA,
        pltpu.SemaphoreType.DMA,
    ],
)(_sc_body)


def _tc_hist_body(p_ref, t_ref, out_ref, acc_ref):
    first_step = (pl.program_id(0) == 0) & (pl.program_id(1) == 0)
    last_step = ((pl.program_id(0) == NB - 1)
                 & (pl.program_id(1) == TC_H // TC_RB - 1))

    planes = [p_ref[0, c] for c in range(NCLS)]
    vals = planes
    while len(vals) > 1:
        nxt = [jnp.maximum(vals[i], vals[i + 1])
               for i in range(0, len(vals) - 1, 2)]
        if len(vals) % 2:
            nxt.append(vals[-1])
        vals = nxt
    m = vals[0]

    t = t_ref[0]

    @pl.when(first_step)
    def _():
        acc_ref[...] = jnp.zeros_like(acc_ref)

    anyf = None
    for c in range(NCLS):
        eqf = jnp.where(planes[c] == m, 1.0, 0.0).astype(jnp.float32)
        if anyf is None:
            firstf = eqf
            anyf = eqf
        else:
            firstf = eqf - eqf * anyf
            anyf = jnp.maximum(anyf, eqf)
        is_t = t == c
        tcf = jnp.where(is_t, 1.0, 0.0).astype(jnp.float32)
        interf = jnp.where(is_t, firstf, 0.0)
        for h, u in enumerate((firstf, tcf, interf)):
            u8 = jnp.sum(u.reshape(TC_RB // 8, 8, W), axis=0)
            acc_ref[h, c] += u8

    @pl.when(last_step)
    def _():
        for h in range(3):
            for c in range(NCLS):
                out_ref[h, c] = jnp.sum(acc_ref[h, c], axis=0, keepdims=True)


_tc_hist = pl.pallas_call(
    _tc_hist_body,
    grid=(NB, TC_H // TC_RB),
    in_specs=[
        pl.BlockSpec((1, NCLS, TC_RB, W), lambda b, r: (b, 0, r, 0)),
        pl.BlockSpec((1, TC_RB, W), lambda b, r: (b, r, 0)),
    ],
    out_specs=pl.BlockSpec((3, NCLS, 1, W), lambda b, r: (0, 0, 0, 0)),
    out_shape=jax.ShapeDtypeStruct((3, NCLS, 1, W), jnp.float32),
    scratch_shapes=[pltpu.VMEM((3, NCLS, 8, W), jnp.float32)],
)


def _epi_body(parts_ref, acc_ref, out_ref):
    p = parts_ref[...]
    s_sc = jnp.sum(p, axis=0)
    a = acc_ref[...]
    s_tc = jnp.sum(jnp.sum(a, axis=3), axis=2)
    s = s_sc[:, :NCLS] + s_tc
    pred_c = s[0:1, :]
    tgt_c = s[1:2, :]
    inter = s[2:3, :]
    union = pred_c + tgt_c - inter
    valid = union > 0.0
    safe = jnp.where(valid, union, 1.0)
    ious = jnp.where(valid, inter / safe, 0.0)
    n = jnp.sum(valid.astype(jnp.float32))
    mean = jnp.where(n > 0.0, jnp.sum(ious) / jnp.maximum(n, 1.0), 0.0)
    iousp = jnp.concatenate(
        [ious, jnp.zeros((1, 32 - NCLS), jnp.float32)], axis=1)
    lanei = lax.broadcasted_iota(jnp.int32, (1, 32), 1)
    row = jnp.where(lanei == NCLS, mean, iousp)
    out_ref[...] = jnp.broadcast_to(row, (8, 32))


_epilogue = pl.pallas_call(
    _epi_body,
    out_shape=jax.ShapeDtypeStruct((8, 32), jnp.float32),
)


def kernel(preds, target):
    t32 = target.astype(jnp.int32)
    parts = _sc_hist(preds, t32)
    acc = _tc_hist(preds, t32)
    out = _epilogue(parts, acc)
    return out[0, :NCLS + 1]


# speedup vs baseline: 1.0721x; 1.0721x over previous
import functools

import jax
import jax.numpy as jnp
from jax import lax
from jax.experimental import pallas as pl
from jax.experimental.pallas import tpu as pltpu
from jax.experimental.pallas import tpu_sc as plsc

NCLS = 19
NB = 4
H = W = 512
NC, NS, L = 2, 16, 16
NW = NC * NS
ROWS = 4

TC_H = 192
SC_H = H - TC_H

BANDS = 16
WCOL = W // 2
BAND_H = SC_H // BANDS
CHUNKS_PER_B = BAND_H // ROWS
TOTAL_CHUNKS = NB * CHUNKS_PER_B
CHUNK = ROWS * WCOL
STEPS = CHUNK // L

TC_RB = 32


def _sc_body(preds_hbm, target_hbm, out_hbm, pbuf, tbuf, hp, ht, hi, part,
             sp0, sp1, st0, st1):
    wid = lax.axis_index("s") * NC + lax.axis_index("c")
    band = lax.shift_right_logical(wid, 1)
    col0 = (wid & 1) * WCOL
    psems = (sp0, sp1)
    tsems = (st0, st1)

    def _descr(k, j):
        if isinstance(k, int):
            b = k // CHUNKS_PER_B
            rem = k % CHUNKS_PER_B
        else:
            b = lax.shift_right_logical(k * 13, 6)
            rem = k - b * CHUNKS_PER_B
        row0 = TC_H + band * BAND_H + rem * ROWS
        dp = pltpu.make_async_copy(
            preds_hbm.at[b, :, pl.ds(row0, ROWS), pl.ds(col0, WCOL)],
            pbuf.at[j], psems[j])
        dt = pltpu.make_async_copy(
            target_hbm.at[b, pl.ds(row0, ROWS), pl.ds(col0, WCOL)],
            tbuf.at[j], tsems[j])
        return dp, dt

    z = jnp.zeros((L,), jnp.float32)
    for r in range(L):
        for h in (hp, ht, hi):
            h[r, pl.ds(0, L)] = z
            h[r, pl.ds(L, L)] = z

    lane = lax.iota(jnp.int32, L)
    ones = jnp.ones((L,), jnp.float32)

    def _argmax_tree(vals):
        idxs = [jnp.full((L,), c, jnp.int32) for c in range(NCLS)]
        while len(vals) > 1:
            nv, ni = [], []
            for a in range(0, len(vals) - 1, 2):
                g = vals[a + 1] > vals[a]
                nv.append(jnp.where(g, vals[a + 1], vals[a]))
                ni.append(jnp.where(g, idxs[a + 1], idxs[a]))
            if len(vals) % 2:
                nv.append(vals[-1])
                ni.append(idxs[-1])
            vals, idxs = nv, ni
        return idxs[0]

    BLOCK = STEPS // ROWS

    def _compute(j):
        def blk(r, carry):
            vals = [pbuf[j, c, r, pl.ds(0, L)] for c in range(NCLS)]
            tcur = tbuf[j, r, pl.ds(0, L)]
            vals1 = [pbuf[j, c, r, pl.ds(L, L)] for c in range(NCLS)]
            tnxt = tbuf[j, r, pl.ds(L, L)]
            for s in range(BLOCK):
                idx = _argmax_tree(vals)
                eq = idx == tcur
                tprev = tcur
                vals, tcur = vals1, tnxt
                if s + 2 < BLOCK:
                    nb = (s + 2) * L
                    vals1 = [pbuf[j, c, r, pl.ds(nb, L)] for c in range(NCLS)]
                    tnxt = tbuf[j, r, pl.ds(nb, L)]
                plsc.addupdate_scatter(hp, [lane, idx], ones)
                plsc.addupdate_scatter(ht, [lane, tprev], ones)
                plsc.addupdate_scatter(hi, [lane, tprev], ones, mask=eq)
            return carry

        lax.fori_loop(0, ROWS, blk, 0)

    d0p, d0t = _descr(0, 0)
    d0p.start()
    d0t.start()
    d1p, d1t = _descr(1, 1)
    d1p.start()
    d1t.start()

    def pair(p, carry):
        k0 = p * 2
        for j in (0, 1):
            k = k0 + j
            dp, dt = _descr(k, j)
            dp.wait()
            dt.wait()
            _compute(j)

            @pl.when(k + 2 < TOTAL_CHUNKS)
            def _():
                sp, st = _descr(k + 2, j)
                sp.start()
                st.start()
        return carry

    lax.fori_loop(0, TOTAL_CHUNKS // 2, pair, 0)

    for row, h in enumerate((hp, ht, hi)):
        a0 = h[0, pl.ds(0, L)]
        a1 = h[0, pl.ds(L, L)]
        for r in range(1, L):
            a0 = a0 + h[r, pl.ds(0, L)]
            a1 = a1 + h[r, pl.ds(L, L)]
        part[row, pl.ds(0, L)] = a0
        part[row, pl.ds(L, L)] = a1
    pltpu.sync_copy(part, out_hbm.at[wid])


_sc_hist = functools.partial(
    pl.kernel,
    out_type=jax.ShapeDtypeStruct((NW, 3, 32), jnp.float32),
    mesh=plsc.VectorSubcoreMesh(core_axis_name="c", subcore_axis_name="s"),
    compiler_params=pltpu.CompilerParams(
        needs_layout_passes=False, skip_device_barrier=True),
    scratch_types=[
        pltpu.VMEM((2, NCLS, ROWS, WCOL), jnp.float32),
        pltpu.VMEM((2, ROWS, WCOL), jnp.int32),
        pltpu.VMEM((L, 32), jnp.float32),
        pltpu.VMEM((L, 32), jnp.float32),
        pltpu.VMEM((L, 32), jnp.float32),
        pltpu.VMEM((3, 32), jnp.float32),
        pltpu.SemaphoreType.DMA,
        pltpu.SemaphoreType.DMA,
        pltpu.SemaphoreType.DMA,
        pltpu.SemaphoreType.DMA,
    ],
)(_sc_body)


def _tc_hist_body(p_ref, t_ref, out_ref, acc_ref):
    first_step = (pl.program_id(0) == 0) & (pl.program_id(1) == 0)
    last_step = ((pl.program_id(0) == NB - 1)
                 & (pl.program_id(1) == TC_H // TC_RB - 1))

    planes = [p_ref[0, c] for c in range(NCLS)]
    vals = planes
    while len(vals) > 1:
        nxt = [jnp.maximum(vals[i], vals[i + 1])
               for i in range(0, len(vals) - 1, 2)]
        if len(vals) % 2:
            nxt.append(vals[-1])
        vals = nxt
    m = vals[0]

    t = t_ref[0]

    @pl.when(first_step)
    def _():
        acc_ref[...] = jnp.zeros_like(acc_ref)

    anyf = None
    for c in range(NCLS):
        eqf = jnp.where(planes[c] == m, 1.0, 0.0).astype(jnp.float32)
        if anyf is None:
            firstf = eqf
            anyf = eqf
        else:
            firstf = eqf - eqf * anyf
            anyf = jnp.maximum(anyf, eqf)
        is_t = t == c
        tcf = jnp.where(is_t, 1.0, 0.0).astype(jnp.float32)
        interf = jnp.where(is_t, firstf, 0.0)
        for h, u in enumerate((firstf, tcf, interf)):
            u8 = jnp.sum(u.reshape(TC_RB // 8, 8, W), axis=0)
            acc_ref[h, c] += u8

    @pl.when(last_step)
    def _():
        for h in range(3):
            for c in range(NCLS):
                out_ref[h, c] = jnp.sum(acc_ref[h, c], axis=0, keepdims=True)


_tc_hist = pl.pallas_call(
    _tc_hist_body,
    grid=(NB, TC_H // TC_RB),
    in_specs=[
        pl.BlockSpec((1, NCLS, TC_RB, W), lambda b, r: (b, 0, r, 0)),
        pl.BlockSpec((1, TC_RB, W), lambda b, r: (b, r, 0)),
    ],
    out_specs=pl.BlockSpec((3, NCLS, 1, W), lambda b, r: (0, 0, 0, 0)),
    out_shape=jax.ShapeDtypeStruct((3, NCLS, 1, W), jnp.float32),
    scratch_shapes=[pltpu.VMEM((3, NCLS, 8, W), jnp.float32)],
)


def _epi_body(parts_ref, acc_ref, out_ref):
    p = parts_ref[...]
    s_sc = jnp.sum(p, axis=0)
    a = acc_ref[...]
    s_tc = jnp.sum(jnp.sum(a, axis=3), axis=2)
    s = s_sc[:, :NCLS] + s_tc
    pred_c = s[0:1, :]
    tgt_c = s[1:2, :]
    inter = s[2:3, :]
    union = pred_c + tgt_c - inter
    valid = union > 0.0
    safe = jnp.where(valid, union, 1.0)
    ious = jnp.where(valid, inter / safe, 0.0)
    n = jnp.sum(valid.astype(jnp.float32))
    mean = jnp.where(n > 0.0, jnp.sum(ious) / jnp.maximum(n, 1.0), 0.0)
    iousp = jnp.concatenate(
        [ious, jnp.zeros((1, 32 - NCLS), jnp.float32)], axis=1)
    lanei = lax.broadcasted_iota(jnp.int32, (1, 32), 1)
    row = jnp.where(lanei == NCLS, mean, iousp)
    out_ref[...] = jnp.broadcast_to(row, (8, 32))


_epilogue = pl.pallas_call(
    _epi_body,
    out_shape=jax.ShapeDtypeStruct((8, 32), jnp.float32),
)


def kernel(preds, target):
    t32 = target.astype(jnp.int32)
    parts = _sc_hist(preds, t32)
    acc = _tc_hist(preds, t32)
    out = _epilogue(parts, acc)
    return out[0, :NCLS + 1]


# speedup vs baseline: 1.0960x; 1.0223x over previous
import functools

import jax
import jax.numpy as jnp
from jax import lax
from jax.experimental import pallas as pl
from jax.experimental.pallas import tpu as pltpu
from jax.experimental.pallas import tpu_sc as plsc

NCLS = 19
NB = 4
H = W = 512
NC, NS, L = 2, 16, 16
NW = NC * NS
ROWS = 4

TC_H = 192
SC_H = H - TC_H

BANDS = 16
WCOL = W // 2
BAND_H = SC_H // BANDS
CHUNKS_PER_B = BAND_H // ROWS
TOTAL_CHUNKS = NB * CHUNKS_PER_B
CHUNK = ROWS * WCOL
STEPS = CHUNK // L

TC_RB = 64


def _sc_body(preds_hbm, target_hbm, out_hbm, pbuf, tbuf, hp, ht, hi, part,
             sp0, sp1, st0, st1):
    wid = lax.axis_index("s") * NC + lax.axis_index("c")
    band = lax.shift_right_logical(wid, 1)
    col0 = (wid & 1) * WCOL
    psems = (sp0, sp1)
    tsems = (st0, st1)

    def _descr(k, j):
        if isinstance(k, int):
            b = k // CHUNKS_PER_B
            rem = k % CHUNKS_PER_B
        else:
            b = lax.shift_right_logical(k * 13, 6)
            rem = k - b * CHUNKS_PER_B
        row0 = TC_H + band * BAND_H + rem * ROWS
        dp = pltpu.make_async_copy(
            preds_hbm.at[b, :, pl.ds(row0, ROWS), pl.ds(col0, WCOL)],
            pbuf.at[j], psems[j])
        dt = pltpu.make_async_copy(
            target_hbm.at[b, pl.ds(row0, ROWS), pl.ds(col0, WCOL)],
            tbuf.at[j], tsems[j])
        return dp, dt

    z = jnp.zeros((L,), jnp.float32)
    for r in range(L):
        for h in (hp, ht, hi):
            h[r, pl.ds(0, L)] = z
            h[r, pl.ds(L, L)] = z

    lane = lax.iota(jnp.int32, L)
    ones = jnp.ones((L,), jnp.float32)

    def _argmax_tree(vals):
        idxs = [jnp.full((L,), c, jnp.int32) for c in range(NCLS)]
        while len(vals) > 1:
            nv, ni = [], []
            for a in range(0, len(vals) - 1, 2):
                g = vals[a + 1] > vals[a]
                nv.append(jnp.where(g, vals[a + 1], vals[a]))
                ni.append(jnp.where(g, idxs[a + 1], idxs[a]))
            if len(vals) % 2:
                nv.append(vals[-1])
                ni.append(idxs[-1])
            vals, idxs = nv, ni
        return idxs[0]

    BLOCK = STEPS // ROWS

    def _compute(j):
        def blk(r, carry):
            vals = [pbuf[j, c, r, pl.ds(0, L)] for c in range(NCLS)]
            tcur = tbuf[j, r, pl.ds(0, L)]
            vals1 = [pbuf[j, c, r, pl.ds(L, L)] for c in range(NCLS)]
            tnxt = tbuf[j, r, pl.ds(L, L)]
            for s in range(BLOCK):
                idx = _argmax_tree(vals)
                eq = idx == tcur
                tprev = tcur
                vals, tcur = vals1, tnxt
                if s + 2 < BLOCK:
                    nb = (s + 2) * L
                    vals1 = [pbuf[j, c, r, pl.ds(nb, L)] for c in range(NCLS)]
                    tnxt = tbuf[j, r, pl.ds(nb, L)]
                plsc.addupdate_scatter(hp, [lane, idx], ones)
                plsc.addupdate_scatter(ht, [lane, tprev], ones)
                plsc.addupdate_scatter(hi, [lane, tprev], ones, mask=eq)
            return carry

        lax.fori_loop(0, ROWS, blk, 0)

    d0p, d0t = _descr(0, 0)
    d0p.start()
    d0t.start()
    d1p, d1t = _descr(1, 1)
    d1p.start()
    d1t.start()

    def pair(p, carry):
        k0 = p * 2
        for j in (0, 1):
            k = k0 + j
            dp, dt = _descr(k, j)
            dp.wait()
            dt.wait()
            _compute(j)

            @pl.when(k + 2 < TOTAL_CHUNKS)
            def _():
                sp, st = _descr(k + 2, j)
                sp.start()
                st.start()
        return carry

    lax.fori_loop(0, TOTAL_CHUNKS // 2, pair, 0)

    for row, h in enumerate((hp, ht, hi)):
        a0 = h[0, pl.ds(0, L)]
        a1 = h[0, pl.ds(L, L)]
        for r in range(1, L):
            a0 = a0 + h[r, pl.ds(0, L)]
            a1 = a1 + h[r, pl.ds(L, L)]
        part[row, pl.ds(0, L)] = a0
        part[row, pl.ds(L, L)] = a1
    pltpu.sync_copy(part, out_hbm.at[wid])


_sc_hist = functools.partial(
    pl.kernel,
    out_type=jax.ShapeDtypeStruct((NW, 3, 32), jnp.float32),
    mesh=plsc.VectorSubcoreMesh(core_axis_name="c", subcore_axis_name="s"),
    compiler_params=pltpu.CompilerParams(
        needs_layout_passes=False, skip_device_barrier=True),
    scratch_types=[
        pltpu.VMEM((2, NCLS, ROWS, WCOL), jnp.float32),
        pltpu.VMEM((2, ROWS, WCOL), jnp.int32),
        pltpu.VMEM((L, 32), jnp.float32),
        pltpu.VMEM((L, 32), jnp.float32),
        pltpu.VMEM((L, 32), jnp.float32),
        pltpu.VMEM((3, 32), jnp.float32),
        pltpu.SemaphoreType.DMA,
        pltpu.SemaphoreType.DMA,
        pltpu.SemaphoreType.DMA,
        pltpu.SemaphoreType.DMA,
    ],
)(_sc_body)


def _tc_hist_body(p_ref, t_ref, out_ref, acc_ref):
    first_step = (pl.program_id(0) == 0) & (pl.program_id(1) == 0)
    last_step = ((pl.program_id(0) == NB - 1)
                 & (pl.program_id(1) == TC_H // TC_RB - 1))

    planes = [p_ref[0, c] for c in range(NCLS)]
    vals = planes
    while len(vals) > 1:
        nxt = [jnp.maximum(vals[i], vals[i + 1])
               for i in range(0, len(vals) - 1, 2)]
        if len(vals) % 2:
            nxt.append(vals[-1])
        vals = nxt
    m = vals[0]

    t = t_ref[0]

    @pl.when(first_step)
    def _():
        acc_ref[...] = jnp.zeros_like(acc_ref)

    anyf = None
    for c in range(NCLS):
        eqf = jnp.where(planes[c] == m, 1.0, 0.0).astype(jnp.float32)
        if anyf is None:
            firstf = eqf
            anyf = eqf
        else:
            firstf = eqf - eqf * anyf
            anyf = jnp.maximum(anyf, eqf)
        is_t = t == c
        tcf = jnp.where(is_t, 1.0, 0.0).astype(jnp.float32)
        interf = jnp.where(is_t, firstf, 0.0)
        for h, u in enumerate((firstf, tcf, interf)):
            u8 = jnp.sum(u.reshape(TC_RB // 8, 8, W), axis=0)
            acc_ref[h, c] += u8

    @pl.when(last_step)
    def _():
        for h in range(3):
            for c in range(NCLS):
                out_ref[h, c] = jnp.sum(acc_ref[h, c], axis=0, keepdims=True)


_tc_hist = pl.pallas_call(
    _tc_hist_body,
    grid=(NB, TC_H // TC_RB),
    in_specs=[
        pl.BlockSpec((1, NCLS, TC_RB, W), lambda b, r: (b, 0, r, 0)),
        pl.BlockSpec((1, TC_RB, W), lambda b, r: (b, r, 0)),
    ],
    out_specs=pl.BlockSpec((3, NCLS, 1, W), lambda b, r: (0, 0, 0, 0)),
    out_shape=jax.ShapeDtypeStruct((3, NCLS, 1, W), jnp.float32),
    scratch_shapes=[pltpu.VMEM((3, NCLS, 8, W), jnp.float32)],
)


def _epi_body(parts_ref, acc_ref, out_ref):
    p = parts_ref[...]
    s_sc = jnp.sum(p, axis=0)
    a = acc_ref[...]
    s_tc = jnp.sum(jnp.sum(a, axis=3), axis=2)
    s = s_sc[:, :NCLS] + s_tc
    pred_c = s[0:1, :]
    tgt_c = s[1:2, :]
    inter = s[2:3, :]
    union = pred_c + tgt_c - inter
    valid = union > 0.0
    safe = jnp.where(valid, union, 1.0)
    ious = jnp.where(valid, inter / safe, 0.0)
    n = jnp.sum(valid.astype(jnp.float32))
    mean = jnp.where(n > 0.0, jnp.sum(ious) / jnp.maximum(n, 1.0), 0.0)
    iousp = jnp.concatenate(
        [ious, jnp.zeros((1, 32 - NCLS), jnp.float32)], axis=1)
    lanei = lax.broadcasted_iota(jnp.int32, (1, 32), 1)
    row = jnp.where(lanei == NCLS, mean, iousp)
    out_ref[...] = row[0, :NCLS + 1]


_epilogue = pl.pallas_call(
    _epi_body,
    out_shape=jax.ShapeDtypeStruct((NCLS + 1,), jnp.float32),
)


def kernel(preds, target):
    t32 = target.astype(jnp.int32)
    parts = _sc_hist(preds, t32)
    acc = _tc_hist(preds, t32)
    return _epilogue(parts, acc)
